# Initial kernel scaffold; baseline (speedup 1.0000x reference)
import jax
import jax.numpy as jnp
from jax.experimental import pallas as pl


def kernel(embedding, W0, b0, g0, be0, W1, b1, g1, be1, edge_index, index):
    raise NotImplementedError("write your pallas kernel here")


# speedup vs baseline: 4.7830x; 4.7830x over previous
import functools

import jax
import jax.numpy as jnp
from jax import lax
from jax.experimental import pallas as pl
from jax.experimental.pallas import tpu as pltpu
from jax.experimental.pallas import tpu_sc as plsc

_NC = 2
_NS = 16
_NW = _NC * _NS
_L = 16


def _pick_chunk(e_per_w):
    for c in range(128, 7, -8):
        if e_per_w % c == 0:
            return c
    raise ValueError(f"no valid chunk for {e_per_w}")


def _make_agg(N, D, E, with_deg):
    assert E % _NW == 0
    e_per_w = E // _NW
    C = _pick_chunk(e_per_w)
    n_chunks = e_per_w // C
    rps = ((N + _NS - 1) // _NS + 7) // 8 * 8
    rlast = N - rps * (_NS - 1)
    assert 0 < rlast <= rps

    mesh = plsc.VectorSubcoreMesh(core_axis_name="c", subcore_axis_name="s")
    out_type = [jax.ShapeDtypeStruct((_NC, N, D), jnp.float32)]
    scratch = [
        pltpu.VMEM((C,), jnp.int32),
        pltpu.VMEM((C,), jnp.int32),
        pltpu.VMEM((C, D), jnp.float32),
        pltpu.VMEM_SHARED((N, D), jnp.float32),
        pltpu.SemaphoreType.DMA,
    ]
    if with_deg:
        out_type.append(jax.ShapeDtypeStruct((_NC, N, _L), jnp.float32))
        scratch += [
            pltpu.VMEM((C, _L), jnp.float32),
            pltpu.VMEM_SHARED((N, _L), jnp.float32),
        ]

    if with_deg:
        def body(feats, srcl, dstl, znd, zdg, onesh,
                 out_acc, out_deg, src_v, dst_v, rows_v, acc_sh, sem,
                 ones_v, deg_sh):
            c = lax.axis_index("c")
            s = lax.axis_index("s")
            wid = c * _NS + s
            r0 = s * rps

            def _zero(sz):
                pltpu.sync_copy(znd.at[pl.ds(r0, sz)],
                                acc_sh.at[pl.ds(r0, sz)])
                pltpu.sync_copy(zdg.at[pl.ds(r0, sz)],
                                deg_sh.at[pl.ds(r0, sz)])

            @pl.when(s < _NS - 1)
            def _():
                _zero(rps)

            @pl.when(s == _NS - 1)
            def _():
                _zero(rlast)

            pltpu.sync_copy(onesh, ones_v)
            plsc.subcore_barrier()
            base0 = wid * e_per_w

            @pl.loop(0, n_chunks)
            def _(i):
                b = base0 + i * C
                pltpu.sync_copy(srcl.at[pl.ds(b, C)], src_v)
                pltpu.sync_copy(dstl.at[pl.ds(b, C)], dst_v)
                pltpu.async_copy(feats.at[src_v], rows_v, sem).wait()
                pltpu.sync_copy(rows_v, acc_sh.at[dst_v], add=True)
                pltpu.sync_copy(ones_v, deg_sh.at[dst_v], add=True)

            plsc.subcore_barrier()

            def _wb(sz):
                pltpu.sync_copy(acc_sh.at[pl.ds(r0, sz)],
                                out_acc.at[c, pl.ds(r0, sz)])
                pltpu.sync_copy(deg_sh.at[pl.ds(r0, sz)],
                                out_deg.at[c, pl.ds(r0, sz)])

            @pl.when(s < _NS - 1)
            def _():
                _wb(rps)

            @pl.when(s == _NS - 1)
            def _():
                _wb(rlast)
    else:
        def body(feats, srcl, dstl, znd,
                 out_acc, src_v, dst_v, rows_v, acc_sh, sem):
            c = lax.axis_index("c")
            s = lax.axis_index("s")
            wid = c * _NS + s
            r0 = s * rps

            @pl.when(s < _NS - 1)
            def _():
                pltpu.sync_copy(znd.at[pl.ds(r0, rps)],
                                acc_sh.at[pl.ds(r0, rps)])

            @pl.when(s == _NS - 1)
            def _():
                pltpu.sync_copy(znd.at[pl.ds(r0, rlast)],
                                acc_sh.at[pl.ds(r0, rlast)])

            plsc.subcore_barrier()
            base0 = wid * e_per_w

            @pl.loop(0, n_chunks)
            def _(i):
                b = base0 + i * C
                pltpu.sync_copy(srcl.at[pl.ds(b, C)], src_v)
                pltpu.sync_copy(dstl.at[pl.ds(b, C)], dst_v)
                pltpu.async_copy(feats.at[src_v], rows_v, sem).wait()
                pltpu.sync_copy(rows_v, acc_sh.at[dst_v], add=True)

            plsc.subcore_barrier()

            @pl.when(s < _NS - 1)
            def _():
                pltpu.sync_copy(acc_sh.at[pl.ds(r0, rps)],
                                out_acc.at[c, pl.ds(r0, rps)])

            @pl.when(s == _NS - 1)
            def _():
                pltpu.sync_copy(acc_sh.at[pl.ds(r0, rlast)],
                                out_acc.at[c, pl.ds(r0, rlast)])

    return pl.kernel(body, out_type=tuple(out_type), mesh=mesh,
                     scratch_types=tuple(scratch)), C


def _make_deg(N, D, E):
    assert E % _NW == 0
    e_per_w = E // _NW
    C = _pick_chunk(e_per_w)
    n_chunks = e_per_w // C
    rps = ((N + _NS - 1) // _NS + 7) // 8 * 8
    rlast = N - rps * (_NS - 1)

    mesh = plsc.VectorSubcoreMesh(core_axis_name="c", subcore_axis_name="s")

    def body(dstl, znd, onesh, out_deg, dst_v, ones_v, deg_sh):
        c = lax.axis_index("c")
        s = lax.axis_index("s")
        wid = c * _NS + s
        r0 = s * rps

        @pl.when(s < _NS - 1)
        def _():
            pltpu.sync_copy(znd.at[pl.ds(r0, rps)],
                            deg_sh.at[pl.ds(r0, rps)])

        @pl.when(s == _NS - 1)
        def _():
            pltpu.sync_copy(znd.at[pl.ds(r0, rlast)],
                            deg_sh.at[pl.ds(r0, rlast)])

        pltpu.sync_copy(onesh, ones_v)
        plsc.subcore_barrier()
        base0 = wid * e_per_w

        @pl.loop(0, n_chunks)
        def _(i):
            b = base0 + i * C
            pltpu.sync_copy(dstl.at[pl.ds(b, C)], dst_v)
            pltpu.sync_copy(ones_v, deg_sh.at[dst_v], add=True)

        plsc.subcore_barrier()

        @pl.when(s < _NS - 1)
        def _():
            pltpu.sync_copy(deg_sh.at[pl.ds(r0, rps)],
                            out_deg.at[c, pl.ds(r0, rps)])

        @pl.when(s == _NS - 1)
        def _():
            pltpu.sync_copy(deg_sh.at[pl.ds(r0, rlast)],
                            out_deg.at[c, pl.ds(r0, rlast)])

    return pl.kernel(
        body,
        out_type=jax.ShapeDtypeStruct((_NC, N, D), jnp.float32),
        mesh=mesh,
        scratch_types=(
            pltpu.VMEM((C,), jnp.int32),
            pltpu.VMEM((C, D), jnp.float32),
            pltpu.VMEM_SHARED((N, D), jnp.float32),
        ),
    ), C


def _deg_call(dst, N, D):
    E = dst.shape[0]
    kern, C = _make_deg(N, D, E)
    znd = jnp.zeros((N, D), jnp.float32)
    onesh = jnp.ones((C, D), jnp.float32)
    return kern(dst, znd, onesh)


def _agg_call(feats, src, dst, with_deg, ones_rows=None):
    N, D = feats.shape
    E = src.shape[0]
    kern, C = _make_agg(N, D, E, with_deg)
    znd = jnp.zeros((N, D), jnp.float32)
    if with_deg:
        zdg = jnp.zeros((N, _L), jnp.float32)
        onesh = jnp.ones((C, _L), jnp.float32)
        return kern(feats, src, dst, znd, zdg, onesh)
    return kern(feats, src, dst, znd)


def _dense_call(acc, deg, feats, W, b, g, be):
    N, D = feats.shape
    bm = max(d for d in range(8, min(N, 1024) + 1, 8) if N % d == 0)
    grid = (N // bm,)

    def body(acc_ref, deg_ref, feats_ref, w_ref, b_ref, g_ref, be_ref,
             out_ref):
        a = acc_ref[0] + acc_ref[1] + feats_ref[...]
        d = deg_ref[0, :, 0:1] + deg_ref[1, :, 0:1]
        x = a / (d + 1.0)
        h = jnp.dot(x, w_ref[...], preferred_element_type=jnp.float32)
        h = h + b_ref[...]
        mu = jnp.mean(h, axis=1, keepdims=True)
        xc = h - mu
        var = jnp.mean(xc * xc, axis=1, keepdims=True)
        y = xc * lax.rsqrt(var + 1e-5) * g_ref[...] + be_ref[...]
        out_ref[...] = jnp.where(y > 0, y, jnp.exp(jnp.minimum(y, 0.0)) - 1.0)

    return pl.pallas_call(
        body,
        grid=grid,
        in_specs=[
            pl.BlockSpec((_NC, bm, D), lambda i: (0, i, 0)),
            pl.BlockSpec((_NC, bm, D), lambda i: (0, i, 0)),
            pl.BlockSpec((bm, D), lambda i: (i, 0)),
            pl.BlockSpec((D, D), lambda i: (0, 0)),
            pl.BlockSpec((1, D), lambda i: (0, 0)),
            pl.BlockSpec((1, D), lambda i: (0, 0)),
            pl.BlockSpec((1, D), lambda i: (0, 0)),
        ],
        out_specs=pl.BlockSpec((bm, D), lambda i: (i, 0)),
        out_shape=jax.ShapeDtypeStruct((N, D), jnp.float32),
    )(acc, deg, feats, W, b.reshape(1, D), g.reshape(1, D), be.reshape(1, D))


def _gather_call(table, idx):
    N, D = table.shape
    B = idx.shape[0]
    assert B % _NW == 0
    bpw = B // _NW
    mesh = plsc.VectorSubcoreMesh(core_axis_name="c", subcore_axis_name="s")

    @functools.partial(
        pl.kernel,
        out_type=jax.ShapeDtypeStruct((B, D), jnp.float32),
        mesh=mesh,
        scratch_types=(
            pltpu.VMEM((bpw,), jnp.int32),
            pltpu.VMEM((bpw, D), jnp.float32),
            pltpu.SemaphoreType.DMA,
        ),
    )
    def k(tbl, idxh, out, idx_v, rows_v, sem):
        wid = lax.axis_index("c") * _NS + lax.axis_index("s")
        base = wid * bpw
        pltpu.sync_copy(idxh.at[pl.ds(base, bpw)], idx_v)
        pltpu.async_copy(tbl.at[idx_v], rows_v, sem).wait()
        pltpu.sync_copy(rows_v, out.at[pl.ds(base, bpw)])

    return k(table, idx)


def kernel(embedding, W0, b0, g0, be0, W1, b1, g1, be1, edge_index, index):
    src = edge_index[0].astype(jnp.int32)
    dst = edge_index[1].astype(jnp.int32)
    idx = index.astype(jnp.int32)
    feats = embedding.astype(jnp.float32)

    N, D = feats.shape

    deg = _deg_call(dst, N, D)
    (acc1,) = _agg_call(feats, src, dst, with_deg=False)
    feats1 = _dense_call(acc1, deg, feats, W0, b0, g0, be0)
    (acc2,) = _agg_call(feats1, src, dst, with_deg=False)
    feats2 = _dense_call(acc2, deg, feats1, W1, b1, g1, be1)
    return _gather_call(feats2, idx)


# speedup vs baseline: 8.2930x; 1.7338x over previous
import functools

import jax
import jax.numpy as jnp
from jax import lax
from jax.experimental import pallas as pl
from jax.experimental.pallas import tpu as pltpu
from jax.experimental.pallas import tpu_sc as plsc

_NC = 2
_NS = 16
_NW = _NC * _NS
_C = 128
_PAD = 8


def _row_split(Np):
    rps = ((Np + _NS - 1) // _NS + 7) // 8 * 8
    rlast = Np - rps * (_NS - 1)
    assert 0 < rlast <= rps
    return rps, rlast


def _mesh():
    return plsc.VectorSubcoreMesh(core_axis_name="c", subcore_axis_name="s")


def _make_agg(N, D, n_chunks):
    Np = N + _PAD
    rps, rlast = _row_split(Np)
    assert n_chunks % 4 == 0
    nh = n_chunks // 2

    def body(feats, src3, dst3, znd, order_dep, out_acc,
             src_all, dst_all, rb0, rb1, acc_sh,
             gs0, gs1, ss0, ss1):
        del order_dep
        rows = (rb0, rb1)
        gsem = (gs0, gs1)
        ssem = (ss0, ss1)
        c = lax.axis_index("c")
        s = lax.axis_index("s")
        wid = c * _NS + s
        r0 = s * rps

        @pl.when(s < _NS - 1)
        def _():
            pltpu.sync_copy(znd.at[pl.ds(r0, rps)],
                            acc_sh.at[pl.ds(r0, rps)])

        @pl.when(s == _NS - 1)
        def _():
            pltpu.sync_copy(znd.at[pl.ds(r0, rlast)],
                            acc_sh.at[pl.ds(r0, rlast)])

        plsc.subcore_barrier()

        def issue_gather(j, b):
            pltpu.async_copy(feats.at[src_all.at[j]], rows[b], gsem[b])

        def wait_gather(j, b):
            pltpu.make_async_copy(feats.at[src_all.at[j]], rows[b],
                                  gsem[b]).wait()

        def issue_scatter(j, b):
            pltpu.async_copy(rows[b], acc_sh.at[dst_all.at[j]], ssem[b],
                             add=True)

        def wait_scatter(j, b):
            pltpu.make_async_copy(rows[b], acc_sh.at[dst_all.at[j]],
                                  ssem[b]).wait()

        def step(j, b):
            wait_gather(j, b)
            issue_scatter(j, b)
            wait_scatter(j - 1, 1 - b)
            issue_gather(j + 1, 1 - b)

        for h in range(2):
            pltpu.sync_copy(src3.at[wid, pl.ds(h * nh, nh)], src_all)
            pltpu.sync_copy(dst3.at[wid, pl.ds(h * nh, nh)], dst_all)
            issue_gather(0, 0)
            wait_gather(0, 0)
            issue_scatter(0, 0)
            issue_gather(1, 1)

            @pl.loop(0, nh // 2 - 1)
            def _(i):
                step(2 * i + 1, 1)
                step(2 * i + 2, 0)

            wait_gather(nh - 1, 1)
            issue_scatter(nh - 1, 1)
            wait_scatter(nh - 2, 0)
            wait_scatter(nh - 1, 1)

        plsc.subcore_barrier()

        @pl.when(s < _NS - 1)
        def _():
            pltpu.sync_copy(acc_sh.at[pl.ds(r0, rps)],
                            out_acc.at[c, pl.ds(r0, rps)])

        @pl.when(s == _NS - 1)
        def _():
            pltpu.sync_copy(acc_sh.at[pl.ds(r0, rlast)],
                            out_acc.at[c, pl.ds(r0, rlast)])

    return pl.kernel(
        body,
        out_type=jax.ShapeDtypeStruct((_NC, Np, D), jnp.float32),
        mesh=_mesh(),
        scratch_types=(
            pltpu.VMEM((n_chunks // 2, _C), jnp.int32),
            pltpu.VMEM((n_chunks // 2, _C), jnp.int32),
            pltpu.VMEM((_C, D), jnp.float32),
            pltpu.VMEM((_C, D), jnp.float32),
            pltpu.VMEM_SHARED((Np, D), jnp.float32),
        ) + (pltpu.SemaphoreType.DMA,) * 4,
    )


def _make_deg(N, D, n_chunks):
    Np = N + _PAD
    rps, rlast = _row_split(Np)
    assert n_chunks % 8 == 0

    def body(dst3, znd, onesh, out_deg, dst_all, ones_v, deg_sh, sem):
        c = lax.axis_index("c")
        s = lax.axis_index("s")
        wid = c * _NS + s
        r0 = s * rps

        @pl.when(s < _NS - 1)
        def _():
            pltpu.sync_copy(znd.at[pl.ds(r0, rps)],
                            deg_sh.at[pl.ds(r0, rps)])

        @pl.when(s == _NS - 1)
        def _():
            pltpu.sync_copy(znd.at[pl.ds(r0, rlast)],
                            deg_sh.at[pl.ds(r0, rlast)])

        pltpu.sync_copy(dst3.at[wid], dst_all)
        pltpu.sync_copy(onesh, ones_v)
        plsc.subcore_barrier()

        @pl.loop(0, n_chunks // 8)
        def _(i):
            j0 = i * 8
            descs = [
                pltpu.async_copy(ones_v, deg_sh.at[dst_all.at[j0 + k]],
                                 sem, add=True)
                for k in range(8)
            ]
            for d in descs:
                d.wait()

        plsc.subcore_barrier()

        @pl.when(s < _NS - 1)
        def _():
            pltpu.sync_copy(deg_sh.at[pl.ds(r0, rps)],
                            out_deg.at[c, pl.ds(r0, rps)])

        @pl.when(s == _NS - 1)
        def _():
            pltpu.sync_copy(deg_sh.at[pl.ds(r0, rlast)],
                            out_deg.at[c, pl.ds(r0, rlast)])

    return pl.kernel(
        body,
        out_type=jax.ShapeDtypeStruct((_NC, Np, D), jnp.float32),
        mesh=_mesh(),
        scratch_types=(
            pltpu.VMEM((n_chunks, _C), jnp.int32),
            pltpu.VMEM((_C, D), jnp.float32),
            pltpu.VMEM_SHARED((Np, D), jnp.float32),
            pltpu.SemaphoreType.DMA,
        ),
    )


def _dense_call(acc, deg, feats, W, b, g, be):
    N, D = feats.shape
    bm = max(d for d in range(8, min(N, 1024) + 1, 8) if N % d == 0)
    grid = (N // bm,)

    def body(acc_ref, deg_ref, feats_ref, w_ref, b_ref, g_ref, be_ref,
             out_ref):
        a = acc_ref[0] + acc_ref[1] + feats_ref[...]
        d = deg_ref[0, :, 0:1] + deg_ref[1, :, 0:1]
        x = a / (d + 1.0)
        h = jnp.dot(x, w_ref[...], preferred_element_type=jnp.float32)
        h = h + b_ref[...]
        mu = jnp.mean(h, axis=1, keepdims=True)
        xc = h - mu
        var = jnp.mean(xc * xc, axis=1, keepdims=True)
        y = xc * lax.rsqrt(var + 1e-5) * g_ref[...] + be_ref[...]
        out_ref[...] = jnp.where(y > 0, y, jnp.exp(jnp.minimum(y, 0.0)) - 1.0)

    return pl.pallas_call(
        body,
        grid=grid,
        in_specs=[
            pl.BlockSpec((_NC, bm, D), lambda i: (0, i, 0)),
            pl.BlockSpec((_NC, bm, D), lambda i: (0, i, 0)),
            pl.BlockSpec((bm, D), lambda i: (i, 0)),
            pl.BlockSpec((D, D), lambda i: (0, 0)),
            pl.BlockSpec((1, D), lambda i: (0, 0)),
            pl.BlockSpec((1, D), lambda i: (0, 0)),
            pl.BlockSpec((1, D), lambda i: (0, 0)),
        ],
        out_specs=pl.BlockSpec((bm, D), lambda i: (i, 0)),
        out_shape=jax.ShapeDtypeStruct((N, D), jnp.float32),
    )(acc, deg, feats, W, b.reshape(1, D), g.reshape(1, D), be.reshape(1, D))


def _gather_call(table, idx):
    N, D = table.shape
    B = idx.shape[0]
    assert B % _NW == 0
    bpw = B // _NW

    @functools.partial(
        pl.kernel,
        out_type=jax.ShapeDtypeStruct((B, D), jnp.float32),
        mesh=_mesh(),
        scratch_types=(
            pltpu.VMEM((bpw,), jnp.int32),
            pltpu.VMEM((bpw, D), jnp.float32),
            pltpu.SemaphoreType.DMA,
        ),
    )
    def k(tbl, idxh, out, idx_v, rows_v, sem):
        wid = lax.axis_index("c") * _NS + lax.axis_index("s")
        base = wid * bpw
        pltpu.sync_copy(idxh.at[pl.ds(base, bpw)], idx_v)
        pltpu.async_copy(tbl.at[idx_v], rows_v, sem).wait()
        pltpu.sync_copy(rows_v, out.at[pl.ds(base, bpw)])

    return k(table, idx)


def kernel(embedding, W0, b0, g0, be0, W1, b1, g1, be1, edge_index, index):
    src = edge_index[0].astype(jnp.int32)
    dst = edge_index[1].astype(jnp.int32)
    idx = index.astype(jnp.int32)
    feats = embedding.astype(jnp.float32)
    N, D = feats.shape
    E = src.shape[0]
    Np = N + _PAD

    assert E % _NW == 0
    e_per_w = E // _NW
    n_chunks = (-(-e_per_w // _C) + 7) // 8 * 8
    assert n_chunks // 4 >= 3
    ppw = n_chunks * _C - e_per_w
    spread = jnp.arange(ppw, dtype=jnp.int32) % _PAD
    src3 = jnp.concatenate(
        [src.reshape(_NW, e_per_w),
         jnp.broadcast_to(spread, (_NW, ppw))], axis=1,
    ).reshape(_NW, n_chunks, _C)
    dst3 = jnp.concatenate(
        [dst.reshape(_NW, e_per_w),
         jnp.broadcast_to(N + spread, (_NW, ppw))], axis=1,
    ).reshape(_NW, n_chunks, _C)

    znd = jnp.zeros((Np, D), jnp.float32)
    onesh = jnp.ones((_C, D), jnp.float32)

    agg = _make_agg(N, D, n_chunks)
    deg = _make_deg(N, D, n_chunks)(dst3, znd, onesh)
    acc1 = agg(feats, src3, dst3, znd, deg)
    feats1 = _dense_call(acc1, deg, feats, W0, b0, g0, be0)
    acc2 = agg(feats1, src3, dst3, znd, feats1)
    feats2 = _dense_call(acc2, deg, feats1, W1, b1, g1, be1)
    return _gather_call(feats2, idx)
